# baseline (speedup 1.0000x reference)
import functools

import jax
import jax.numpy as jnp
from jax import lax
from jax.experimental import pallas as pl
from jax.experimental.pallas import tpu as pltpu
from jax.experimental.pallas import tpu_sc as plsc

ORIG_VOCAB = 1000000
NEW_VOCAB = 1000
D = 64
L = 16
NC, NS = 2, 16
NW = NC * NS
HIST = 200
HALF_OFF = (0, 72)


def _sc_body(ids_hbm, orig_hbm, new_hbm, out_hbm, newtbl_v, idx_v, cid_v,
             rows_v, sem):
    wid = lax.axis_index("s") * NC + lax.axis_index("c")
    batch = ids_hbm.shape[0]
    rows_per_w = batch // NW
    base_b = wid * rows_per_w

    pltpu.sync_copy(new_hbm, newtbl_v)

    def row_body(i, carry):
        b = base_b + i
        pltpu.sync_copy(ids_hbm.at[b], idx_v)

        for h in range(2):
            for g in range(128 // L):
                off = HALF_OFF[h] + g * L
                v = idx_v[pl.ds(off, L)]
                cid_v[h, pl.ds(g * L, L)] = jnp.minimum(v, ORIG_VOCAB - 1)

        d0 = pltpu.async_copy(orig_hbm.at[cid_v.at[0]],
                              rows_v.at[pl.ds(0, 128)], sem)
        d1 = pltpu.async_copy(orig_hbm.at[cid_v.at[1]],
                              rows_v.at[pl.ds(HALF_OFF[1], 128)], sem)
        d0.wait()
        d1.wait()

        for h in range(2):
            for g in range(128 // L):
                off = HALF_OFF[h] + g * L
                v = idx_v[pl.ds(off, L)]
                m = v >= ORIG_VOCAB
                cnt = plsc.all_reduce_population_count(m)[0]

                @pl.when(cnt > 0)
                def _():
                    nid = jnp.where(m, v - ORIG_VOCAB, 0)
                    lrow = lax.iota(jnp.int32, L) + off
                    for c in range(D):
                        col = jnp.full((L,), c, jnp.int32)
                        vals = plsc.load_gather(newtbl_v, [nid, col])
                        plsc.store_scatter(rows_v, [lrow, col], vals, mask=m)

        pltpu.sync_copy(rows_v, out_hbm.at[b])
        return carry

    lax.fori_loop(0, rows_per_w, row_body, 0)


@functools.lru_cache(maxsize=None)
def _make_sc_call(batch, hist):
    mesh = plsc.VectorSubcoreMesh(core_axis_name="c", subcore_axis_name="s")
    return pl.kernel(
        _sc_body,
        out_type=jax.ShapeDtypeStruct((batch, hist, D), jnp.float32),
        mesh=mesh,
        scratch_types=[
            pltpu.VMEM((NEW_VOCAB, D), jnp.float32),
            pltpu.VMEM((HIST,), jnp.int32),
            pltpu.VMEM((2, 128), jnp.int32),
            pltpu.VMEM((HIST, D), jnp.float32),
            pltpu.SemaphoreType.DMA,
        ],
        compiler_params=pltpu.CompilerParams(
            use_tc_tiling_on_sc=False, needs_layout_passes=False),
    )


@jax.jit
def kernel(input_ids, orig_table, new_table):
    b, h = input_ids.shape
    ids = input_ids.astype(jnp.int32)
    return _make_sc_call(b, h)(ids, orig_table, new_table)
